# baseline (speedup 1.0000x reference)
import functools

import jax
import jax.numpy as jnp
import numpy as np
from jax import lax
from jax.experimental import pallas as pl
from jax.experimental.pallas import tpu as pltpu
from jax.experimental.pallas import tpu_sc as plsc

N_HEAD = 4
Q_DIM = 8
DEPTH = 8
V_DIM = 8
NV = 2 ** DEPTH
NK = NV - 1

NC = 2
NS = 16
NW = NC * NS
L = 16

CHUNK = 256
EINV = np.float32(np.exp(-1.0))
LN2 = np.float32(np.log(2.0))


def _tree_body(x_hbm, q_hbm, u1_hbm, u2_hbm, c_hbm, uv_hbm, s1_hbm, s2_hbm,
               vt_hbm, svt_hbm, vb_hbm, vs_hbm,
               x_v, q_v, u1_v, u2_v, c_v, uv_v, s1_v, s2_v, vt_v, svt_v,
               vb_v, vs_v, sem_t, sem_a, sem_b):
    wid = lax.axis_index("s") * NC + lax.axis_index("c")
    h1 = pltpu.async_copy(s1_hbm, s1_v, sem_t)
    h2 = pltpu.async_copy(s2_hbm, s2_v, sem_t)
    h3 = pltpu.async_copy(vt_hbm, vt_v, sem_t)
    h4 = pltpu.async_copy(svt_hbm, svt_v, sem_t)

    iota = lax.broadcasted_iota(jnp.int32, (L,), 0)
    ixh = iota & (N_HEAD - 1)
    ib4 = iota >> 2

    rows_per_worker = c_hbm.shape[1] // NW
    nchunks = rows_per_worker // CHUNK
    sems = (sem_a, sem_b)

    h5 = pltpu.async_copy(
        q_hbm.at[:, pl.ds(wid * (rows_per_worker // N_HEAD),
                          rows_per_worker // N_HEAD)], q_v, sem_t)

    def fire(ch, s):
        base = wid * rows_per_worker + ch * CHUNK
        sem = sems[s]
        return [
            pltpu.async_copy(x_hbm.at[:, pl.ds(base, CHUNK)], x_v.at[s], sem),
            pltpu.async_copy(u1_hbm.at[:, pl.ds(base, CHUNK)], u1_v.at[s],
                             sem),
            pltpu.async_copy(u2_hbm.at[:, pl.ds(base, CHUNK)], u2_v.at[s],
                             sem),
            pltpu.async_copy(c_hbm.at[:, pl.ds(base, CHUNK)], c_v.at[s], sem),
            pltpu.async_copy(uv_hbm.at[:, pl.ds(base, CHUNK)], uv_v.at[s],
                             sem),
        ]

    def make_group_body(s, ch):
        qch = ch * (CHUNK // N_HEAD)

        def group_body(g):
            r = g * L
            qs_raw = [x_v[s, j, pl.ds(r, L)] for j in range(Q_DIM)]
            qsel = [plsc.load_gather(
                        q_v, [jnp.full((L,), j, jnp.int32),
                              ib4 + (qch + g * (L // N_HEAD))])
                    for j in range(Q_DIM)]
            xs = [w * (np.float32(2.0) * qv - np.float32(1.0))
                  for w, qv in zip(qs_raw, qsel)]
            exs = [jnp.exp(v) for v in xs]
            enxs = [np.float32(1.0) / v for v in exs]
            ix = jnp.zeros((L,), jnp.int32)
            T = jnp.zeros((L,), jnp.float32)
            for d in range(DEPTH):
                node = ixh * NK + (2 ** d - 1) + ix
                S = jnp.zeros((L,), jnp.float32)
                for j in range(Q_DIM):
                    idx = node + j * (N_HEAD * NK)
                    s1 = plsc.load_gather(s1_v, [idx])
                    s2 = plsc.load_gather(s2_v, [idx])
                    u1 = u1_v[s, d * Q_DIM + j, pl.ds(r, L)]
                    u2 = u2_v[s, d * Q_DIM + j, pl.ds(r, L)]
                    A = jnp.where(u1 < s1, EINV, np.float32(1.0)) + exs[j]
                    Bv = jnp.where(u2 < s2, EINV, np.float32(1.0)) + enxs[j]
                    S = S + A * Bv / (A + Bv)
                c = c_v[s, d, pl.ds(r, L)]
                lor = S < c
                ix = 2 * ix + lor.astype(jnp.int32)
                T = T + jnp.where(lor, S, np.float32(1.0) / S)
            node_v = ixh * NV + ix
            for j in range(V_DIM):
                idx = node_v + j * (N_HEAD * NV)
                vw = plsc.load_gather(vt_v, [idx])
                sv = plsc.load_gather(svt_v, [idx])
                uv = uv_v[s, j, pl.ds(r, L)]
                vb = uv < sv
                vsp = jnp.where(vb, vw, -vw)
                y = jnp.exp(-vsp) + T
                yi = lax.bitcast_convert_type(y, jnp.int32)
                k = (yi >> 23) - 127
                m = lax.bitcast_convert_type((yi & 0x7FFFFF) | 0x3F800000,
                                             jnp.float32)
                adj = m > np.float32(1.5)
                m = jnp.where(adj, m * np.float32(0.5), m)
                kf = (k + adj.astype(jnp.int32)).astype(jnp.float32)
                sm = (m - np.float32(1.0)) / (m + np.float32(1.0))
                s2q = sm * sm
                lnm = np.float32(2.0) * sm * (
                    np.float32(1.0) + s2q * (
                        np.float32(1.0 / 3) + s2q * (
                            np.float32(1.0 / 5) + s2q * (
                                np.float32(1.0 / 7)
                                + s2q * np.float32(1.0 / 9)))))
                vs_v[j, pl.ds(r, L)] = -(lnm + kf * LN2)
                vb_v[j, pl.ds(r, L)] = jnp.where(vb, np.float32(1.0),
                                                 np.float32(0.0))
        return group_body

    h1.wait(); h2.wait(); h3.wait(); h4.wait(); h5.wait()
    handles = fire(0, 0)
    for ch in range(nchunks):
        s = ch % 2
        nxt = fire(ch + 1, 1 - s) if ch + 1 < nchunks else None
        for h in handles:
            h.wait()
        plsc.parallel_loop(0, CHUNK // L)(make_group_body(s, ch))
        base = wid * rows_per_worker + ch * CHUNK
        pltpu.sync_copy(vb_v, vb_hbm.at[:, pl.ds(base, CHUNK)])
        pltpu.sync_copy(vs_v, vs_hbm.at[:, pl.ds(base, CHUNK)])
        handles = nxt


@functools.cache
def _rng_constants(B: int):
    try:
        with jax.ensure_compile_time_eval():
            out = _rng_constants_impl(B)
            return tuple(jax.block_until_ready(o) for o in out)
    except Exception:
        return None


def _rng_constants_impl(B):
    H = N_HEAD
    Q = Q_DIM
    BH = B * H
    rk = jax.random.key(42)
    us = jnp.stack([
        jax.random.uniform(jax.random.fold_in(rk, d), (BH, 2 * Q), jnp.float32)
        for d in range(DEPTH)])
    u1T = us[:, :, 0::2].transpose(0, 2, 1).reshape(DEPTH * Q, BH)
    u2T = us[:, :, 1::2].transpose(0, 2, 1).reshape(DEPTH * Q, BH)
    ul = jnp.stack([
        jax.random.uniform(jax.random.fold_in(rk, 100 + d), (BH,), jnp.float32)
        for d in range(DEPTH)])
    cT = (1.0 - ul) / ul
    uvT = jax.random.uniform(jax.random.fold_in(rk, 999), (B, H, V_DIM),
                             jnp.float32).reshape(BH, V_DIM).T
    return (u1T, u2T, cT, uvT)


def kernel(q, qs, key_param, value_param):
    B, Q = q.shape
    H = N_HEAD
    BH = B * H

    xT = qs.T
    qT = q.astype(jnp.float32).T

    consts = _rng_constants(B)
    if consts is None:
        consts = _rng_constants_impl(B)
    u1T, u2T, cT, uvT = consts

    key_flat = key_param.reshape(-1, 2 * Q)
    sk = jax.nn.sigmoid(key_flat)
    s1t = sk[:, 0::2].T.reshape(-1)
    s2t = sk[:, 1::2].T.reshape(-1)
    value_flat = value_param.reshape(-1, V_DIM)
    vt = value_flat.T.reshape(-1)
    svt = jax.nn.sigmoid(value_flat).T.reshape(-1)

    f32 = jnp.float32
    mesh = plsc.VectorSubcoreMesh(core_axis_name="c", subcore_axis_name="s")
    vbT, vsT = pl.kernel(
        _tree_body,
        out_type=[jax.ShapeDtypeStruct((V_DIM, BH), f32),
                  jax.ShapeDtypeStruct((V_DIM, BH), f32)],
        mesh=mesh,
        compiler_params=pltpu.CompilerParams(needs_layout_passes=False),
        scratch_types=[
            pltpu.VMEM((2, Q_DIM, CHUNK), f32),
            pltpu.VMEM((Q_DIM, 1024 // N_HEAD), f32),
            pltpu.VMEM((2, DEPTH * Q_DIM, CHUNK), f32),
            pltpu.VMEM((2, DEPTH * Q_DIM, CHUNK), f32),
            pltpu.VMEM((2, DEPTH, CHUNK), f32),
            pltpu.VMEM((2, V_DIM, CHUNK), f32),
            pltpu.VMEM((Q_DIM * H * NK,), f32),
            pltpu.VMEM((Q_DIM * H * NK,), f32),
            pltpu.VMEM((V_DIM * H * NV,), f32),
            pltpu.VMEM((V_DIM * H * NV,), f32),
            pltpu.VMEM((V_DIM, CHUNK), f32),
            pltpu.VMEM((V_DIM, CHUNK), f32),
            pltpu.SemaphoreType.DMA,
            pltpu.SemaphoreType.DMA,
            pltpu.SemaphoreType.DMA,
        ],
    )(xT, qT, u1T, u2T, cT, uvT, s1t, s2t, vt, svt)

    vb = jnp.transpose(vbT.reshape(V_DIM, B, H), (1, 2, 0)).astype(bool)
    value_s = jnp.transpose(vsT.reshape(V_DIM, B, H), (1, 2, 0))
    return (vb, value_s)


# speedup vs baseline: 1.0435x; 1.0435x over previous
import functools

import jax
import jax.numpy as jnp
import numpy as np
from jax import lax
from jax.experimental import pallas as pl
from jax.experimental.pallas import tpu as pltpu
from jax.experimental.pallas import tpu_sc as plsc

N_HEAD = 4
Q_DIM = 8
DEPTH = 8
V_DIM = 8
NV = 2 ** DEPTH
NK = NV - 1

NC = 2
NS = 16
NW = NC * NS
L = 16

CHUNK = 256
EINV = np.float32(np.exp(-1.0))
LN2 = np.float32(np.log(2.0))


def _tree_body(x_hbm, q_hbm, u1_hbm, u2_hbm, c_hbm, uv_hbm, s1_hbm, s2_hbm,
               vt_hbm, svt_hbm, vb_hbm, vs_hbm,
               x_v, q_v, u1_v, u2_v, c_v, uv_v, s1_v, s2_v, vt_v, svt_v,
               vb_v, vs_v, sem_t, sem_a, sem_b):
    wid = lax.axis_index("s") * NC + lax.axis_index("c")
    h1 = pltpu.async_copy(s1_hbm, s1_v, sem_t)
    h2 = pltpu.async_copy(s2_hbm, s2_v, sem_t)
    h3 = pltpu.async_copy(vt_hbm, vt_v, sem_t)
    h4 = pltpu.async_copy(svt_hbm, svt_v, sem_t)

    iota = lax.broadcasted_iota(jnp.int32, (L,), 0)
    ixh = iota & (N_HEAD - 1)
    ib4 = iota >> 2

    rows_per_worker = c_hbm.shape[1] // NW
    nchunks = rows_per_worker // CHUNK
    sems = (sem_a, sem_b)

    h5 = pltpu.async_copy(
        q_hbm.at[:, pl.ds(wid * (rows_per_worker // N_HEAD),
                          rows_per_worker // N_HEAD)], q_v, sem_t)

    def fire(ch, s):
        base = wid * rows_per_worker + ch * CHUNK
        sem = sems[s]
        return [
            pltpu.async_copy(x_hbm.at[:, pl.ds(base, CHUNK)], x_v.at[s], sem),
            pltpu.async_copy(u1_hbm.at[:, pl.ds(base, CHUNK)], u1_v.at[s],
                             sem),
            pltpu.async_copy(u2_hbm.at[:, pl.ds(base, CHUNK)], u2_v.at[s],
                             sem),
            pltpu.async_copy(c_hbm.at[:, pl.ds(base, CHUNK)], c_v.at[s], sem),
            pltpu.async_copy(uv_hbm.at[:, pl.ds(base, CHUNK)], uv_v.at[s],
                             sem),
        ]

    def make_group_body(s, ch):
        qch = ch * (CHUNK // N_HEAD)

        def group_body(g):
            r = g * L
            qs_raw = [x_v[s, j, pl.ds(r, L)] for j in range(Q_DIM)]
            qsel = [plsc.load_gather(
                        q_v, [jnp.full((L,), j, jnp.int32),
                              ib4 + (qch + g * (L // N_HEAD))])
                    for j in range(Q_DIM)]
            xs = [w * (np.float32(2.0) * qv - np.float32(1.0))
                  for w, qv in zip(qs_raw, qsel)]
            exs = [jnp.exp(v) for v in xs]
            enxs = [np.float32(1.0) / v for v in exs]

            def depth_body(d, carry):
                ix, T = carry
                off = lax.shift_left(jnp.int32(1), d) - 1
                node = ixh * NK + off + ix
                S = jnp.zeros((L,), jnp.float32)
                for j in range(Q_DIM):
                    idx = node + j * (N_HEAD * NK)
                    s1 = plsc.load_gather(s1_v, [idx])
                    s2 = plsc.load_gather(s2_v, [idx])
                    u1 = u1_v[s, d * Q_DIM + j, pl.ds(r, L)]
                    u2 = u2_v[s, d * Q_DIM + j, pl.ds(r, L)]
                    A = jnp.where(u1 < s1, EINV, np.float32(1.0)) + exs[j]
                    Bv = jnp.where(u2 < s2, EINV, np.float32(1.0)) + enxs[j]
                    S = S + A * Bv / (A + Bv)
                c = c_v[s, d, pl.ds(r, L)]
                lor = S < c
                ix = 2 * ix + lor.astype(jnp.int32)
                T = T + jnp.where(lor, S, np.float32(1.0) / S)
                return (ix, T)

            ix, T = lax.fori_loop(
                0, DEPTH, depth_body,
                (jnp.zeros((L,), jnp.int32), jnp.zeros((L,), jnp.float32)))
            node_v = ixh * NV + ix

            def value_body(j, carry):
                idx = node_v + j * (N_HEAD * NV)
                vw = plsc.load_gather(vt_v, [idx])
                sv = plsc.load_gather(svt_v, [idx])
                uv = uv_v[s, j, pl.ds(r, L)]
                vb = uv < sv
                vsp = jnp.where(vb, vw, -vw)
                y = jnp.exp(-vsp) + T
                yi = lax.bitcast_convert_type(y, jnp.int32)
                k = (yi >> 23) - 127
                m = lax.bitcast_convert_type((yi & 0x7FFFFF) | 0x3F800000,
                                             jnp.float32)
                adj = m > np.float32(1.5)
                m = jnp.where(adj, m * np.float32(0.5), m)
                kf = (k + adj.astype(jnp.int32)).astype(jnp.float32)
                sm = (m - np.float32(1.0)) / (m + np.float32(1.0))
                s2q = sm * sm
                lnm = np.float32(2.0) * sm * (
                    np.float32(1.0) + s2q * (
                        np.float32(1.0 / 3) + s2q * (
                            np.float32(1.0 / 5) + s2q * (
                                np.float32(1.0 / 7)
                                + s2q * np.float32(1.0 / 9)))))
                vs_v[j, pl.ds(r, L)] = -(lnm + kf * LN2)
                vb_v[j, pl.ds(r, L)] = jnp.where(vb, np.float32(1.0),
                                                 np.float32(0.0))
                return carry

            lax.fori_loop(0, V_DIM, value_body, jnp.int32(0))
        return group_body

    h1.wait(); h2.wait(); h3.wait(); h4.wait(); h5.wait()
    handles = fire(0, 0)
    for ch in range(nchunks):
        s = ch % 2
        nxt = fire(ch + 1, 1 - s) if ch + 1 < nchunks else None
        for h in handles:
            h.wait()
        plsc.parallel_loop(0, CHUNK // L)(make_group_body(s, ch))
        base = wid * rows_per_worker + ch * CHUNK
        pltpu.sync_copy(vb_v, vb_hbm.at[:, pl.ds(base, CHUNK)])
        pltpu.sync_copy(vs_v, vs_hbm.at[:, pl.ds(base, CHUNK)])
        handles = nxt


@functools.cache
def _rng_constants(B: int):
    try:
        with jax.ensure_compile_time_eval():
            out = _rng_constants_impl(B)
            return tuple(jax.block_until_ready(o) for o in out)
    except Exception:
        return None


def _rng_constants_impl(B):
    H = N_HEAD
    Q = Q_DIM
    BH = B * H
    rk = jax.random.key(42)
    us = jnp.stack([
        jax.random.uniform(jax.random.fold_in(rk, d), (BH, 2 * Q), jnp.float32)
        for d in range(DEPTH)])
    u1T = us[:, :, 0::2].transpose(0, 2, 1).reshape(DEPTH * Q, BH)
    u2T = us[:, :, 1::2].transpose(0, 2, 1).reshape(DEPTH * Q, BH)
    ul = jnp.stack([
        jax.random.uniform(jax.random.fold_in(rk, 100 + d), (BH,), jnp.float32)
        for d in range(DEPTH)])
    cT = (1.0 - ul) / ul
    uvT = jax.random.uniform(jax.random.fold_in(rk, 999), (B, H, V_DIM),
                             jnp.float32).reshape(BH, V_DIM).T
    return (u1T, u2T, cT, uvT)


def kernel(q, qs, key_param, value_param):
    B, Q = q.shape
    H = N_HEAD
    BH = B * H

    xT = qs.T
    qT = q.astype(jnp.float32).T

    consts = _rng_constants(B)
    if consts is None:
        consts = _rng_constants_impl(B)
    u1T, u2T, cT, uvT = consts

    key_flat = key_param.reshape(-1, 2 * Q)
    sk = jax.nn.sigmoid(key_flat)
    s1t = sk[:, 0::2].T.reshape(-1)
    s2t = sk[:, 1::2].T.reshape(-1)
    value_flat = value_param.reshape(-1, V_DIM)
    vt = value_flat.T.reshape(-1)
    svt = jax.nn.sigmoid(value_flat).T.reshape(-1)

    f32 = jnp.float32
    mesh = plsc.VectorSubcoreMesh(core_axis_name="c", subcore_axis_name="s")
    vbT, vsT = pl.kernel(
        _tree_body,
        out_type=[jax.ShapeDtypeStruct((V_DIM, BH), f32),
                  jax.ShapeDtypeStruct((V_DIM, BH), f32)],
        mesh=mesh,
        compiler_params=pltpu.CompilerParams(needs_layout_passes=False),
        scratch_types=[
            pltpu.VMEM((2, Q_DIM, CHUNK), f32),
            pltpu.VMEM((Q_DIM, 1024 // N_HEAD), f32),
            pltpu.VMEM((2, DEPTH * Q_DIM, CHUNK), f32),
            pltpu.VMEM((2, DEPTH * Q_DIM, CHUNK), f32),
            pltpu.VMEM((2, DEPTH, CHUNK), f32),
            pltpu.VMEM((2, V_DIM, CHUNK), f32),
            pltpu.VMEM((Q_DIM * H * NK,), f32),
            pltpu.VMEM((Q_DIM * H * NK,), f32),
            pltpu.VMEM((V_DIM * H * NV,), f32),
            pltpu.VMEM((V_DIM * H * NV,), f32),
            pltpu.VMEM((V_DIM, CHUNK), f32),
            pltpu.VMEM((V_DIM, CHUNK), f32),
            pltpu.SemaphoreType.DMA,
            pltpu.SemaphoreType.DMA,
            pltpu.SemaphoreType.DMA,
        ],
    )(xT, qT, u1T, u2T, cT, uvT, s1t, s2t, vt, svt)

    vb = jnp.transpose(vbT.reshape(V_DIM, B, H), (1, 2, 0)).astype(bool)
    value_s = jnp.transpose(vsT.reshape(V_DIM, B, H), (1, 2, 0))
    return (vb, value_s)
